# baseline (speedup 1.0000x reference)
import functools

import jax
import jax.numpy as jnp
from jax import lax
from jax.experimental import pallas as pl
from jax.experimental.pallas import tpu as pltpu
from jax.experimental.pallas import tpu_sc as plsc

_BATCH = 16384
_HIST = 200
_D = 32
_L = 16

_info = plsc.get_sparse_core_info()
_NC, _NS = _info.num_cores, _info.num_subcores
_NW = _NC * _NS
_B = 512
_BLK_PER_H = _BATCH // _B
_NCHUNK = _HIST * _BLK_PER_H // _NW


def _make_kernel():
    mesh = plsc.VectorSubcoreMesh(core_axis_name="c", subcore_axis_name="s")

    @functools.partial(
        pl.kernel,
        mesh=mesh,
        out_type=jax.ShapeDtypeStruct((_HIST, _D, _BATCH), jnp.float32),
        scratch_types=[
            pltpu.VMEM((2, _B), jnp.int32),
            pltpu.VMEM((2, _B, _D), jnp.float32),
            pltpu.VMEM((2, _D, _B), jnp.float32),
            pltpu.SemaphoreType.DMA((2,)),
            pltpu.SemaphoreType.DMA((2,)),
            pltpu.SemaphoreType.DMA((2,)),
        ],
        compiler_params=pltpu.CompilerParams(
            use_tc_tiling_on_sc=False, needs_layout_passes=False
        ),
    )
    def body(idxt_hbm, w_hbm, out_hbm, idx_v, rows_v, trans_v,
             sem_i, sem_g, sem_o):
        wid = lax.axis_index("s") * _NC + lax.axis_index("c")
        t0 = wid * _NCHUNK

        def hb(t):
            if isinstance(t, int):
                return t // _BLK_PER_H, (t % _BLK_PER_H) * _B
            return lax.div(t, _BLK_PER_H), lax.rem(t, _BLK_PER_H) * _B

        def bsl(g):
            return g % 2 if isinstance(g, int) else lax.rem(g, 2)

        def start_idx(g):
            b = bsl(g)
            h, b0 = hb(t0 + g)
            pltpu.async_copy(
                idxt_hbm.at[h, pl.ds(b0, _B)], idx_v.at[b], sem_i.at[b]
            )

        def wait_idx(g):
            b = bsl(g)
            pltpu.make_async_copy(
                idxt_hbm.at[0, pl.ds(0, _B)], idx_v.at[b], sem_i.at[b]
            ).wait()

        def start_gather(g):
            b = bsl(g)
            pltpu.async_copy(w_hbm.at[idx_v.at[b]], rows_v.at[b], sem_g.at[b])

        def wait_gather(g):
            b = bsl(g)
            pltpu.make_async_copy(
                w_hbm.at[pl.ds(0, _B)], rows_v.at[b], sem_g.at[b]
            ).wait()

        def transpose(g):
            b = bsl(g)

            def blk_body(blk, carry):
                row_ids = blk * _L + lax.iota(jnp.int32, _L)
                for f in range(_D):
                    col_ids = jnp.full((_L,), f, jnp.int32)
                    vals = plsc.load_gather(rows_v.at[b], [row_ids, col_ids])
                    trans_v[b, f, pl.ds(blk * _L, _L)] = vals
                return carry

            lax.fori_loop(0, _B // _L, blk_body, 0)

        def start_store(g):
            b = bsl(g)
            h, b0 = hb(t0 + g)
            pltpu.async_copy(
                trans_v.at[b], out_hbm.at[h, :, pl.ds(b0, _B)], sem_o.at[b]
            )

        def wait_store(g):
            b = bsl(g)
            pltpu.make_async_copy(
                trans_v.at[b], out_hbm.at[0, :, pl.ds(0, _B)], sem_o.at[b]
            ).wait()

        start_idx(0)
        wait_idx(0)
        start_gather(0)
        start_idx(1)

        def step(g, carry):
            wait_gather(g - 1)

            @pl.when(g + 1 < _NCHUNK)
            def _():
                start_idx(g + 1)

            wait_idx(g)
            start_gather(g)
            transpose(g - 1)

            @pl.when(g >= 2)
            def _():
                wait_store(g - 2)

            start_store(g - 1)
            return carry

        lax.fori_loop(1, _NCHUNK, step, 0)

        wait_gather(_NCHUNK - 1)
        transpose(_NCHUNK - 1)
        wait_store(_NCHUNK - 2)
        start_store(_NCHUNK - 1)
        wait_store(_NCHUNK - 1)

    return body


_gather_call = _make_kernel()


def kernel(input_tensor, weight):
    out_t = _gather_call(input_tensor.T, weight)
    return jnp.transpose(out_t, (2, 0, 1))


# speedup vs baseline: 1.0373x; 1.0373x over previous
import functools

import jax
import jax.numpy as jnp
from jax import lax
from jax.experimental import pallas as pl
from jax.experimental.pallas import tpu as pltpu
from jax.experimental.pallas import tpu_sc as plsc

_BATCH = 16384
_HIST = 200
_D = 32

_info = plsc.get_sparse_core_info()
_NC, _NS = _info.num_cores, _info.num_subcores
_NW = _NC * _NS
_B = 1024
_BLK_PER_H = _BATCH // _B
_NCHUNK = _HIST * _BLK_PER_H // _NW


def _make_gather():
    mesh = plsc.VectorSubcoreMesh(core_axis_name="c", subcore_axis_name="s")

    @functools.partial(
        pl.kernel,
        mesh=mesh,
        out_type=jax.ShapeDtypeStruct((_HIST, _BATCH, _D), jnp.float32),
        scratch_types=[
            pltpu.VMEM((2, _B), jnp.int32),
            pltpu.VMEM((2, _B, _D), jnp.float32),
            pltpu.SemaphoreType.DMA((2,)),
            pltpu.SemaphoreType.DMA((2,)),
            pltpu.SemaphoreType.DMA((2,)),
        ],
        compiler_params=pltpu.CompilerParams(use_tc_tiling_on_sc=False),
    )
    def body(idxt_hbm, w_hbm, out_hbm, idx_v, rows_v, sem_i, sem_g, sem_o):
        wid = lax.axis_index("s") * _NC + lax.axis_index("c")
        t0 = wid * _NCHUNK

        def hb(t):
            if isinstance(t, int):
                return t // _BLK_PER_H, (t % _BLK_PER_H) * _B
            return lax.div(t, _BLK_PER_H), lax.rem(t, _BLK_PER_H) * _B

        def bsl(g):
            return g % 2 if isinstance(g, int) else lax.rem(g, 2)

        def start_idx(g):
            b = bsl(g)
            h, b0 = hb(t0 + g)
            pltpu.async_copy(
                idxt_hbm.at[h, pl.ds(b0, _B)], idx_v.at[b], sem_i.at[b]
            )

        def wait_idx(g):
            b = bsl(g)
            pltpu.make_async_copy(
                idxt_hbm.at[0, pl.ds(0, _B)], idx_v.at[b], sem_i.at[b]
            ).wait()

        def start_gather(g):
            b = bsl(g)
            pltpu.async_copy(w_hbm.at[idx_v.at[b]], rows_v.at[b], sem_g.at[b])

        def wait_gather(g):
            b = bsl(g)
            pltpu.make_async_copy(
                w_hbm.at[pl.ds(0, _B)], rows_v.at[b], sem_g.at[b]
            ).wait()

        def start_store(g):
            b = bsl(g)
            h, b0 = hb(t0 + g)
            pltpu.async_copy(
                rows_v.at[b], out_hbm.at[h, pl.ds(b0, _B)], sem_o.at[b]
            )

        def wait_store(g):
            b = bsl(g)
            pltpu.make_async_copy(
                rows_v.at[b], out_hbm.at[0, pl.ds(0, _B)], sem_o.at[b]
            ).wait()

        start_idx(0)
        wait_idx(0)
        start_gather(0)
        start_idx(1)

        def step(g, carry):
            wait_gather(g - 1)

            @pl.when(g + 1 < _NCHUNK)
            def _():
                start_idx(g + 1)

            @pl.when(g >= 2)
            def _():
                wait_store(g - 2)

            wait_idx(g)
            start_gather(g)
            start_store(g - 1)
            return carry

        lax.fori_loop(1, _NCHUNK, step, 0)

        wait_gather(_NCHUNK - 1)
        wait_store(_NCHUNK - 2)
        start_store(_NCHUNK - 1)
        wait_store(_NCHUNK - 1)

    return body


_gather_call = _make_gather()

_TB = 2048


def _tbody(x_ref, o_ref):
    o_ref[...] = jnp.swapaxes(x_ref[...], 1, 2)


def _tc_transpose(x):
    return pl.pallas_call(
        _tbody,
        grid=(_HIST, _BATCH // _TB),
        in_specs=[pl.BlockSpec((1, _TB, _D), lambda h, j: (h, j, 0))],
        out_specs=pl.BlockSpec((1, _D, _TB), lambda h, j: (h, 0, j)),
        out_shape=jax.ShapeDtypeStruct((_HIST, _D, _BATCH), jnp.float32),
    )(x)


def kernel(input_tensor, weight):
    out_hm = _gather_call(input_tensor.T, weight)
    out_t = _tc_transpose(out_hm)
    return jnp.transpose(out_t, (2, 0, 1))


# speedup vs baseline: 1.2765x; 1.2305x over previous
import functools

import jax
import jax.numpy as jnp
from jax import lax
from jax.experimental import pallas as pl
from jax.experimental.pallas import tpu as pltpu
from jax.experimental.pallas import tpu_sc as plsc

_BATCH = 16384
_HIST = 200
_D = 32
_L = 16

_info = plsc.get_sparse_core_info()
_NC, _NS = _info.num_cores, _info.num_subcores
_NW = _NC * _NS
_B = 512
_BLK_PER_H = _BATCH // _B
_NCHUNK = _HIST * _BLK_PER_H // _NW


def _make_kernel():
    mesh = plsc.VectorSubcoreMesh(core_axis_name="c", subcore_axis_name="s")

    @functools.partial(
        pl.kernel,
        mesh=mesh,
        out_type=jax.ShapeDtypeStruct((_HIST, _D, _BATCH), jnp.float32),
        scratch_types=[
            pltpu.VMEM((2, _B), jnp.int32),
            pltpu.VMEM((2, _B, _D), jnp.float32),
            pltpu.VMEM((2, _D, _B), jnp.float32),
            pltpu.SemaphoreType.DMA((2,)),
            pltpu.SemaphoreType.DMA((2,)),
            pltpu.SemaphoreType.DMA((2,)),
        ],
        compiler_params=pltpu.CompilerParams(
            use_tc_tiling_on_sc=False, needs_layout_passes=False
        ),
    )
    def body(idxt_hbm, w_hbm, out_hbm, idx_v, rows_v, trans_v,
             sem_i, sem_g, sem_o):
        wid = lax.axis_index("s") * _NC + lax.axis_index("c")
        t0 = wid * _NCHUNK

        def hb(t):
            if isinstance(t, int):
                return t // _BLK_PER_H, (t % _BLK_PER_H) * _B
            return lax.div(t, _BLK_PER_H), lax.rem(t, _BLK_PER_H) * _B

        def start_idx(g, par):
            h, b0 = hb(t0 + g)
            pltpu.async_copy(
                idxt_hbm.at[h, pl.ds(b0, _B)], idx_v.at[par], sem_i.at[par]
            )

        def wait_idx(par):
            pltpu.make_async_copy(
                idxt_hbm.at[0, pl.ds(0, _B)], idx_v.at[par], sem_i.at[par]
            ).wait()

        def start_gather(par):
            pltpu.async_copy(
                w_hbm.at[idx_v.at[par]], rows_v.at[par], sem_g.at[par]
            )

        def wait_gather(par):
            pltpu.make_async_copy(
                w_hbm.at[pl.ds(0, _B)], rows_v.at[par], sem_g.at[par]
            ).wait()

        def transpose(par):
            rows = rows_v.at[par]
            base = lax.iota(jnp.int32, _L)

            @plsc.parallel_loop(0, _B // _L, unroll=4)
            def _(blk):
                row_ids = base + blk * _L
                for f in range(_D):
                    col_ids = jnp.full((_L,), f, jnp.int32)
                    vals = plsc.load_gather(rows, [row_ids, col_ids])
                    trans_v[par, f, pl.ds(blk * _L, _L)] = vals

        def start_store(g, par):
            h, b0 = hb(t0 + g)
            pltpu.async_copy(
                trans_v.at[par], out_hbm.at[h, :, pl.ds(b0, _B)], sem_o.at[par]
            )

        def wait_store(par):
            pltpu.make_async_copy(
                trans_v.at[par], out_hbm.at[0, :, pl.ds(0, _B)], sem_o.at[par]
            ).wait()

        def halfstep(g, par):
            wait_gather(1 - par)

            @pl.when(g + 1 < _NCHUNK)
            def _():
                start_idx(g + 1, 1 - par)

            wait_idx(par)
            start_gather(par)
            transpose(1 - par)

            @pl.when(g >= 2)
            def _():
                wait_store(par)

            start_store(g - 1, 1 - par)

        start_idx(0, 0)
        wait_idx(0)
        start_gather(0)
        start_idx(1, 1)

        def pair(p, carry):
            g = 2 * p + 1
            halfstep(g, 1)
            halfstep(g + 1, 0)
            return carry

        lax.fori_loop(0, (_NCHUNK - 2) // 2, pair, 0)

        halfstep(_NCHUNK - 1, 1)
        wait_gather(1)
        transpose(1)
        start_store(_NCHUNK - 1, 1)
        wait_store(0)
        wait_store(1)

    return body


_gather_call = _make_kernel()


def kernel(input_tensor, weight):
    out_t = _gather_call(input_tensor.T, weight)
    return jnp.transpose(out_t, (2, 0, 1))


# speedup vs baseline: 1.3899x; 1.0888x over previous
import functools

import jax
import jax.numpy as jnp
from jax import lax
from jax.experimental import pallas as pl
from jax.experimental.pallas import tpu as pltpu
from jax.experimental.pallas import tpu_sc as plsc

_BATCH = 16384
_HIST = 200
_D = 32
_N = _BATCH * _HIST

_info = plsc.get_sparse_core_info()
_NC, _NS = _info.num_cores, _info.num_subcores
_NW = _NC * _NS
_PER_W = _N // _NW
_C = 1024
_NCHUNK = _PER_W // _C


def _make_kernel():
    mesh = plsc.VectorSubcoreMesh(core_axis_name="c", subcore_axis_name="s")

    @functools.partial(
        pl.kernel,
        mesh=mesh,
        out_type=jax.ShapeDtypeStruct((_N, _D), jnp.float32),
        scratch_types=[
            pltpu.VMEM((2, _C), jnp.int32),
            pltpu.VMEM((2, _C, _D), jnp.float32),
            pltpu.SemaphoreType.DMA((2,)),
            pltpu.SemaphoreType.DMA((2,)),
            pltpu.SemaphoreType.DMA((2,)),
        ],
        compiler_params=pltpu.CompilerParams(
            use_tc_tiling_on_sc=False, needs_layout_passes=False
        ),
    )
    def body(idx_hbm, w_hbm, out_hbm, idx_v, rows_v, sem_i, sem_g, sem_o):
        wid = lax.axis_index("s") * _NC + lax.axis_index("c")
        r0 = wid * _PER_W

        def start_idx(g, par):
            pltpu.async_copy(
                idx_hbm.at[pl.ds(r0 + g * _C, _C)], idx_v.at[par],
                sem_i.at[par],
            )

        def wait_idx(par):
            pltpu.make_async_copy(
                idx_hbm.at[pl.ds(0, _C)], idx_v.at[par], sem_i.at[par]
            ).wait()

        def start_gather(par):
            pltpu.async_copy(
                w_hbm.at[idx_v.at[par]], rows_v.at[par], sem_g.at[par]
            )

        def wait_gather(par):
            pltpu.make_async_copy(
                w_hbm.at[pl.ds(0, _C)], rows_v.at[par], sem_g.at[par]
            ).wait()

        def start_store(g, par):
            pltpu.async_copy(
                rows_v.at[par], out_hbm.at[pl.ds(r0 + g * _C, _C)],
                sem_o.at[par],
            )

        def wait_store(par):
            pltpu.make_async_copy(
                rows_v.at[par], out_hbm.at[pl.ds(0, _C)], sem_o.at[par]
            ).wait()

        def step(g, par):
            wait_idx(par)

            @pl.when(g >= 2)
            def _():
                wait_store(par)

            start_gather(par)
            wait_gather(par)

            @pl.when(g + 2 < _NCHUNK)
            def _():
                start_idx(g + 2, par)

            start_store(g, par)

        start_idx(0, 0)
        start_idx(1, 1)

        def pair(p, carry):
            g = 2 * p
            step(g, 0)
            step(g + 1, 1)
            return carry

        lax.fori_loop(0, _NCHUNK // 2, pair, 0)

        wait_store(0)
        wait_store(1)

    return body


_gather_call = _make_kernel()


def kernel(input_tensor, weight):
    out_flat = _gather_call(input_tensor.reshape(_N), weight)
    return out_flat.reshape(_BATCH, _HIST, _D)


# speedup vs baseline: 1.3910x; 1.0008x over previous
import functools

import jax
import jax.numpy as jnp
from jax import lax
from jax.experimental import pallas as pl
from jax.experimental.pallas import tpu as pltpu
from jax.experimental.pallas import tpu_sc as plsc

_BATCH = 16384
_HIST = 200
_D = 32
_N = _BATCH * _HIST

_info = plsc.get_sparse_core_info()
_NC, _NS = _info.num_cores, _info.num_subcores
_NW = _NC * _NS
_PER_W = _N // _NW
_C = 1600
_NCHUNK = _PER_W // _C


def _make_kernel():
    mesh = plsc.VectorSubcoreMesh(core_axis_name="c", subcore_axis_name="s")

    @functools.partial(
        pl.kernel,
        mesh=mesh,
        out_type=jax.ShapeDtypeStruct((_N, _D), jnp.float32),
        scratch_types=[
            pltpu.VMEM((2, _C), jnp.int32),
            pltpu.VMEM((2, _C, _D), jnp.float32),
            pltpu.SemaphoreType.DMA((2,)),
            pltpu.SemaphoreType.DMA((2,)),
            pltpu.SemaphoreType.DMA((2,)),
        ],
        compiler_params=pltpu.CompilerParams(
            use_tc_tiling_on_sc=False, needs_layout_passes=False
        ),
    )
    def body(idx_hbm, w_hbm, out_hbm, idx_v, rows_v, sem_i, sem_g, sem_o):
        wid = lax.axis_index("s") * _NC + lax.axis_index("c")
        r0 = wid * _PER_W

        def start_idx(g, par):
            pltpu.async_copy(
                idx_hbm.at[pl.ds(r0 + g * _C, _C)], idx_v.at[par],
                sem_i.at[par],
            )

        def wait_idx(par):
            pltpu.make_async_copy(
                idx_hbm.at[pl.ds(0, _C)], idx_v.at[par], sem_i.at[par]
            ).wait()

        def start_gather(par):
            pltpu.async_copy(
                w_hbm.at[idx_v.at[par]], rows_v.at[par], sem_g.at[par]
            )

        def wait_gather(par):
            pltpu.make_async_copy(
                w_hbm.at[pl.ds(0, _C)], rows_v.at[par], sem_g.at[par]
            ).wait()

        def start_store(g, par):
            pltpu.async_copy(
                rows_v.at[par], out_hbm.at[pl.ds(r0 + g * _C, _C)],
                sem_o.at[par],
            )

        def wait_store(par):
            pltpu.make_async_copy(
                rows_v.at[par], out_hbm.at[pl.ds(0, _C)], sem_o.at[par]
            ).wait()

        def step(g, par):
            wait_idx(par)

            @pl.when(g >= 2)
            def _():
                wait_store(par)

            start_gather(par)
            wait_gather(par)

            @pl.when(g + 2 < _NCHUNK)
            def _():
                start_idx(g + 2, par)

            start_store(g, par)

        start_idx(0, 0)
        start_idx(1, 1)

        def pair(p, carry):
            g = 2 * p
            step(g, 0)
            step(g + 1, 1)
            return carry

        lax.fori_loop(0, _NCHUNK // 2, pair, 0)

        wait_store(0)
        wait_store(1)

    return body


_gather_call = _make_kernel()


def kernel(input_tensor, weight):
    out_flat = _gather_call(input_tensor.reshape(_N), weight)
    return out_flat.reshape(_BATCH, _HIST, _D)


# speedup vs baseline: 1.3949x; 1.0028x over previous
import functools

import jax
import jax.numpy as jnp
from jax import lax
from jax.experimental import pallas as pl
from jax.experimental.pallas import tpu as pltpu
from jax.experimental.pallas import tpu_sc as plsc

_BATCH = 16384
_HIST = 200
_D = 32
_N = _BATCH * _HIST

_info = plsc.get_sparse_core_info()
_NC, _NS = _info.num_cores, _info.num_subcores
_NW = _NC * _NS
_PER_W = _N // _NW
_C = 800
_NB = 4
_NCHUNK = _PER_W // _C


def _make_kernel():
    mesh = plsc.VectorSubcoreMesh(core_axis_name="c", subcore_axis_name="s")

    @functools.partial(
        pl.kernel,
        mesh=mesh,
        out_type=jax.ShapeDtypeStruct((_N, _D), jnp.float32),
        scratch_types=[
            pltpu.VMEM((_NB, _C), jnp.int32),
            pltpu.VMEM((_NB, _C, _D), jnp.float32),
            pltpu.SemaphoreType.DMA((_NB,)),
            pltpu.SemaphoreType.DMA((_NB,)),
            pltpu.SemaphoreType.DMA((_NB,)),
        ],
        compiler_params=pltpu.CompilerParams(
            use_tc_tiling_on_sc=False, needs_layout_passes=False
        ),
    )
    def body(idx_hbm, w_hbm, out_hbm, idx_v, rows_v, sem_i, sem_g, sem_o):
        wid = lax.axis_index("s") * _NC + lax.axis_index("c")
        r0 = wid * _PER_W

        def start_idx(g, par):
            pltpu.async_copy(
                idx_hbm.at[pl.ds(r0 + g * _C, _C)], idx_v.at[par],
                sem_i.at[par],
            )

        def wait_idx(par):
            pltpu.make_async_copy(
                idx_hbm.at[pl.ds(0, _C)], idx_v.at[par], sem_i.at[par]
            ).wait()

        def start_gather(par):
            pltpu.async_copy(
                w_hbm.at[idx_v.at[par]], rows_v.at[par], sem_g.at[par]
            )

        def wait_gather(par):
            pltpu.make_async_copy(
                w_hbm.at[pl.ds(0, _C)], rows_v.at[par], sem_g.at[par]
            ).wait()

        def start_store(g, par):
            pltpu.async_copy(
                rows_v.at[par], out_hbm.at[pl.ds(r0 + g * _C, _C)],
                sem_o.at[par],
            )

        def wait_store(par):
            pltpu.make_async_copy(
                rows_v.at[par], out_hbm.at[pl.ds(0, _C)], sem_o.at[par]
            ).wait()

        def step(g, p):
            q = (p - 1) % _NB
            wait_idx(p)

            @pl.when(g >= _NB)
            def _():
                wait_store(p)

            start_gather(p)
            wait_gather(q)
            start_store(g - 1, q)

            @pl.when(g + _NB - 1 < _NCHUNK)
            def _():
                start_idx(g + _NB - 1, q)

        for b in range(_NB):
            start_idx(b, b)
        wait_idx(0)
        start_gather(0)

        def quad(qd, carry):
            g = _NB * qd + 1
            for k in range(_NB):
                step(g + k, (g + k) % _NB)
            return carry

        lax.fori_loop(0, (_NCHUNK - 1) // _NB, quad, 0)

        for g in range(1 + _NB * ((_NCHUNK - 1) // _NB), _NCHUNK):
            step(g, g % _NB)
        last = (_NCHUNK - 1) % _NB
        wait_gather(last)
        start_store(_NCHUNK - 1, last)
        for b in range(_NB):
            wait_store(b)

    return body


_gather_call = _make_kernel()


def kernel(input_tensor, weight):
    out_flat = _gather_call(input_tensor.reshape(_N), weight)
    return out_flat.reshape(_BATCH, _HIST, _D)
